# Initial kernel scaffold; baseline (speedup 1.0000x reference)
import jax
import jax.numpy as jnp
from jax.experimental import pallas as pl


def kernel(user_embedding, item_embedding, edge_user, edge_item, edge_weight):
    raise NotImplementedError("write your pallas kernel here")


# speedup vs baseline: 3.4165x; 3.4165x over previous
import functools

import jax
import jax.numpy as jnp
from jax import lax
from jax.experimental import pallas as pl
from jax.experimental.pallas import tpu as pltpu
from jax.experimental.pallas import tpu_sc as plsc

_NU = 50000
_NI = 50000
_NP = 50176
_DH = 32
_NE = 800000
_NT = 16
_EPT = 51200
_CH = 128
_SUB = 8
_NSC = _EPT // (_CH * _SUB)
_RPT = _NP // _NT
_DR = 112
_NDR = _RPT // _DR


def _body(uh, ih, eu, ei, ew, uo, io, sua, sub_, sia, sib,
          bi_src, bi_dst, bw, rows, dnew, dprev, zbuf, accum, sem):
    c = lax.axis_index("c")
    s = lax.axis_index("s")
    row0 = s * _RPT

    def fill_z(r, carry):
        zbuf[r, pl.ds(0, 16)] = jnp.zeros((16,), jnp.float32)
        zbuf[r, pl.ds(16, 16)] = jnp.zeros((16,), jnp.float32)
        return carry
    lax.fori_loop(0, _DR, fill_z, 0)

    uh_c, ih_c = uh.at[c], ih.at[c]
    uo_c, io_c = uo.at[c], io.at[c]
    sua_c, sub_c = sua.at[c], sub_.at[c]
    sia_c, sib_c = sia.at[c], sib.at[c]

    def run_pass(src, isrc, idst, scratch_dst, prev, out, final):
        def zb(k, carry):
            pltpu.sync_copy(zbuf, accum.at[pl.ds(row0 + k * _DR, _DR)])
            return carry
        lax.fori_loop(0, _NDR, zb, 0)
        plsc.subcore_barrier()

        def superchunk(sc, carry):
            b2 = s * (_EPT // _CH) + sc * _SUB
            pltpu.sync_copy(isrc.at[pl.ds(b2, _SUB)], bi_src)
            pltpu.sync_copy(idst.at[pl.ds(b2, _SUB)], bi_dst)
            pltpu.sync_copy(ew.at[pl.ds(b2 * _CH, _SUB * _CH)], bw)

            def subchunk(j, carry1):
                pltpu.async_copy(src.at[bi_src.at[j]], rows, sem).wait()

                def scale(g, carry2):
                    wv = bw[pl.ds(j * _CH + g * 16, 16)]
                    for e16 in range(16):
                        e = g * 16 + e16
                        w = wv[e16]
                        rows[e, pl.ds(0, 16)] = rows[e, pl.ds(0, 16)] * w
                        rows[e, pl.ds(16, 16)] = rows[e, pl.ds(16, 16)] * w
                    return carry2
                lax.fori_loop(0, _CH // 16, scale, 0)
                pltpu.sync_copy(rows, accum.at[bi_dst.at[j]], add=True)
                return carry1
            lax.fori_loop(0, _SUB, subchunk, 0)
            return carry
        lax.fori_loop(0, _NSC, superchunk, 0)
        plsc.subcore_barrier()

        def dump(k, carry):
            r = row0 + k * _DR
            pltpu.sync_copy(accum.at[pl.ds(r, _DR)], dnew)
            if scratch_dst is not None:
                pltpu.sync_copy(dnew, scratch_dst.at[pl.ds(r, _DR)])
            pltpu.sync_copy(prev.at[pl.ds(r, _DR)], dprev)

            def acc(rr, carry2):
                for h in (0, 16):
                    v = dnew[rr, pl.ds(h, 16)] + dprev[rr, pl.ds(h, 16)]
                    if final:
                        v = v * 0.25
                    dnew[rr, pl.ds(h, 16)] = v
                return carry2
            lax.fori_loop(0, _DR, acc, 0)
            pltpu.sync_copy(dnew, out.at[pl.ds(r, _DR)])
            return carry
        lax.fori_loop(0, _NDR, dump, 0)
        plsc.subcore_barrier()

    run_pass(ih_c, ei, eu, sua_c, uh_c, uo_c, False)
    run_pass(uh_c, eu, ei, sia_c, ih_c, io_c, False)
    run_pass(sia_c, ei, eu, sub_c, uo_c, uo_c, False)
    run_pass(sua_c, eu, ei, sib_c, io_c, io_c, False)
    run_pass(sib_c, ei, eu, None, uo_c, uo_c, True)
    run_pass(sub_c, eu, ei, None, io_c, io_c, True)


@jax.jit
def _run(uh, ih, eu, ei, ew):
    f32 = jnp.float32
    mesh = plsc.VectorSubcoreMesh(core_axis_name="c", subcore_axis_name="s")
    out_type = [
        jax.ShapeDtypeStruct((2, _NP, _DH), f32),
        jax.ShapeDtypeStruct((2, _NP, _DH), f32),
        jax.ShapeDtypeStruct((2, _NP, _DH), f32),
        jax.ShapeDtypeStruct((2, _NP, _DH), f32),
        jax.ShapeDtypeStruct((2, _NP, _DH), f32),
        jax.ShapeDtypeStruct((2, _NP, _DH), f32),
    ]
    scratch = [
        pltpu.VMEM((_SUB, _CH), jnp.int32),
        pltpu.VMEM((_SUB, _CH), jnp.int32),
        pltpu.VMEM((_SUB * _CH,), f32),
        pltpu.VMEM((_CH, _DH), f32),
        pltpu.VMEM((_DR, _DH), f32),
        pltpu.VMEM((_DR, _DH), f32),
        pltpu.VMEM((_DR, _DH), f32),
        pltpu.VMEM_SHARED((_NP, _DH), f32),
        pltpu.SemaphoreType.DMA,
    ]
    fn = pl.kernel(_body, out_type=out_type, mesh=mesh, scratch_types=scratch,
                   compiler_params=pltpu.CompilerParams(use_tc_tiling_on_sc=False))
    return fn(uh, ih, eu, ei, ew)


def kernel(user_embedding, item_embedding, edge_user, edge_item, edge_weight):
    ue = jnp.pad(user_embedding, ((0, _NP - _NU), (0, 0)))
    ie = jnp.pad(item_embedding, ((0, _NP - _NI), (0, 0)))
    uh = jnp.stack([ue[:, :_DH], ue[:, _DH:]])
    ih = jnp.stack([ie[:, :_DH], ie[:, _DH:]])
    per = _NE // _NT
    pad = _EPT - per

    def prep(x):
        return jnp.pad(x.reshape(_NT, per), ((0, 0), (0, pad))).reshape(-1, _CH)

    eu = prep(edge_user)
    ei = prep(edge_item)
    ew = prep(edge_weight).reshape(-1)
    uo, io, *_ = _run(uh, ih, eu, ei, ew)
    embed_user = jnp.concatenate([uo[0, :_NU], uo[1, :_NU]], axis=1)
    embed_item = jnp.concatenate([io[0, :_NI], io[1, :_NI]], axis=1)
    return (embed_user, embed_item)


# speedup vs baseline: 4.7188x; 1.3812x over previous
import jax
import jax.numpy as jnp
from jax import lax
from jax.experimental import pallas as pl
from jax.experimental.pallas import tpu as pltpu
from jax.experimental.pallas import tpu_sc as plsc

_NU = 50000
_NI = 50000
_NP = 50176
_DH = 32
_NE = 800000
_NT = 16
_EPT = 51200
_CH = 128
_SUB = 8
_NSC = _EPT // (_CH * _SUB)
_RPT = _NP // _NT
_DR = 112
_NDR = _RPT // _DR


def _body(uh, ih, edat_u, edat_i, ew, uo, io, sua, sub_, sia, sib,
          ball, bw, rows0, rows1, dnew, dprev, zbuf,
          accum, gs0, gs1, ss0, ss1, zs):
    c = lax.axis_index("c")
    s = lax.axis_index("s")
    row0 = s * _RPT

    def fill_z(r, carry):
        zbuf[r, pl.ds(0, 16)] = jnp.zeros((16,), jnp.float32)
        zbuf[r, pl.ds(16, 16)] = jnp.zeros((16,), jnp.float32)
        return carry
    lax.fori_loop(0, _DR, fill_z, 0)

    uh_c, ih_c = uh.at[c], ih.at[c]
    uo_c, io_c = uo.at[c], io.at[c]
    sua_c, sub_c = sua.at[c], sub_.at[c]
    sia_c, sib_c = sia.at[c], sib.at[c]
    rowbufs = (rows0, rows1)
    gsems = (gs0, gs1)
    ssems = (ss0, ss1)

    def run_pass(src, edat, scratch_dst, prev, out, final):
        zds = [
            pltpu.make_async_copy(
                zbuf, accum.at[pl.ds(row0 + k * _DR, _DR)], zs)
            for k in range(_NDR)
        ]
        for d in zds:
            d.start()
        for d in zds:
            d.wait()
        plsc.subcore_barrier()

        def superchunk(sc, carry):
            pltpu.sync_copy(edat.at[s * _NSC + sc], ball)
            pltpu.sync_copy(
                ew.at[pl.ds((s * _NSC + sc) * _SUB * _CH, _SUB * _CH)], bw)
            gd = [None, None]
            sd = [None, None]
            gd[0] = pltpu.async_copy(
                src.at[ball.at[0].at[0]], rowbufs[0], gsems[0])
            for j in range(_SUB):
                p = j % 2
                q = 1 - p
                if j + 1 < _SUB:
                    if sd[q] is not None:
                        sd[q].wait()
                        sd[q] = None
                    gd[q] = pltpu.async_copy(
                        src.at[ball.at[0].at[j + 1]], rowbufs[q], gsems[q])
                gd[p].wait()
                rb = rowbufs[p]

                def scale(g, carry2, _j=j, _rb=rb):
                    wv = bw[pl.ds(_j * _CH + g * 16, 16)]
                    for e16 in range(16):
                        e = g * 16 + e16
                        w = wv[e16]
                        _rb[e, pl.ds(0, 16)] = _rb[e, pl.ds(0, 16)] * w
                        _rb[e, pl.ds(16, 16)] = _rb[e, pl.ds(16, 16)] * w
                    return carry2
                lax.fori_loop(0, _CH // 16, scale, 0)
                sd[p] = pltpu.async_copy(
                    rb, accum.at[ball.at[1].at[j]], ssems[p], add=True)
            for d in sd:
                if d is not None:
                    d.wait()
            return carry
        lax.fori_loop(0, _NSC, superchunk, 0)
        plsc.subcore_barrier()

        def dump(k, carry):
            r = row0 + k * _DR
            pltpu.sync_copy(accum.at[pl.ds(r, _DR)], dnew)
            if scratch_dst is not None:
                pltpu.sync_copy(dnew, scratch_dst.at[pl.ds(r, _DR)])
            pltpu.sync_copy(prev.at[pl.ds(r, _DR)], dprev)

            def acc(rr, carry2):
                for h in (0, 16):
                    v = dnew[rr, pl.ds(h, 16)] + dprev[rr, pl.ds(h, 16)]
                    if final:
                        v = v * 0.25
                    dnew[rr, pl.ds(h, 16)] = v
                return carry2
            lax.fori_loop(0, _DR, acc, 0)
            pltpu.sync_copy(dnew, out.at[pl.ds(r, _DR)])
            return carry
        lax.fori_loop(0, _NDR, dump, 0)
        plsc.subcore_barrier()

    run_pass(ih_c, edat_u, sua_c, uh_c, uo_c, False)
    run_pass(uh_c, edat_i, sia_c, ih_c, io_c, False)
    run_pass(sia_c, edat_u, sub_c, uo_c, uo_c, False)
    run_pass(sua_c, edat_i, sib_c, io_c, io_c, False)
    run_pass(sib_c, edat_u, None, uo_c, uo_c, True)
    run_pass(sub_c, edat_i, None, io_c, io_c, True)


@jax.jit
def _run(uh, ih, edat_u, edat_i, ew):
    f32 = jnp.float32
    mesh = plsc.VectorSubcoreMesh(core_axis_name="c", subcore_axis_name="s")
    out_type = [
        jax.ShapeDtypeStruct((2, _NP, _DH), f32),
        jax.ShapeDtypeStruct((2, _NP, _DH), f32),
        jax.ShapeDtypeStruct((2, _NP, _DH), f32),
        jax.ShapeDtypeStruct((2, _NP, _DH), f32),
        jax.ShapeDtypeStruct((2, _NP, _DH), f32),
        jax.ShapeDtypeStruct((2, _NP, _DH), f32),
    ]
    scratch = [
        pltpu.VMEM((2, _SUB, _CH), jnp.int32),
        pltpu.VMEM((_SUB * _CH,), f32),
        pltpu.VMEM((_CH, _DH), f32),
        pltpu.VMEM((_CH, _DH), f32),
        pltpu.VMEM((_DR, _DH), f32),
        pltpu.VMEM((_DR, _DH), f32),
        pltpu.VMEM((_DR, _DH), f32),
        pltpu.VMEM_SHARED((_NP, _DH), f32),
        pltpu.SemaphoreType.DMA,
        pltpu.SemaphoreType.DMA,
        pltpu.SemaphoreType.DMA,
        pltpu.SemaphoreType.DMA,
        pltpu.SemaphoreType.DMA,
    ]
    fn = pl.kernel(_body, out_type=out_type, mesh=mesh, scratch_types=scratch,
                   compiler_params=pltpu.CompilerParams(use_tc_tiling_on_sc=False))
    return fn(uh, ih, edat_u, edat_i, ew)


def kernel(user_embedding, item_embedding, edge_user, edge_item, edge_weight):
    ue = jnp.pad(user_embedding, ((0, _NP - _NU), (0, 0)))
    ie = jnp.pad(item_embedding, ((0, _NP - _NI), (0, 0)))
    uh = jnp.stack([ue[:, :_DH], ue[:, _DH:]])
    ih = jnp.stack([ie[:, :_DH], ie[:, _DH:]])
    per = _NE // _NT
    pad = _EPT - per

    def prep(x):
        return jnp.pad(x.reshape(_NT, per), ((0, 0), (0, pad))).reshape(
            _NT * _NSC, _SUB, _CH)

    eu = prep(edge_user)
    ei = prep(edge_item)
    ew = prep(edge_weight).reshape(-1)
    edat_u = jnp.stack([ei, eu], axis=1)
    edat_i = jnp.stack([eu, ei], axis=1)
    uo, io, *_ = _run(uh, ih, edat_u, edat_i, ew)
    embed_user = jnp.concatenate([uo[0, :_NU], uo[1, :_NU]], axis=1)
    embed_item = jnp.concatenate([io[0, :_NI], io[1, :_NI]], axis=1)
    return (embed_user, embed_item)
